# baseline (speedup 1.0000x reference)
import functools

import jax
import jax.numpy as jnp
import numpy as np
from jax import lax
from jax.experimental import pallas as pl
from jax.experimental.pallas import tpu as pltpu
from jax.experimental.pallas import tpu_sc as plsc

D = 32
EPS = 1e-5

NC = 2
NS = 16
NW = NC * NS

IDX_MINOR = 128
GATHERS_PER_CHUNK = 8
CHUNK = IDX_MINOR * GATHERS_PER_CHUNK

TBLK = 8192
TSUB = TBLK // 4

KB = 2048
RB = KB // 4


def _tc_pack_table(table_t, ident, n_blocks):
    def body(z_ref, e_ref, o_ref):
        z = z_ref[...]
        acc = None
        for q in range(4):
            zq = z[:, q * TSUB:(q + 1) * TSUB]
            eq = e_ref[pl.ds(q * D, D), :]
            t = lax.dot_general(
                zq, eq, (((0,), (0,)), ((), ())),
                precision=lax.Precision.HIGHEST,
                preferred_element_type=jnp.float32,
            )
            acc = t if acc is None else acc + t
        o_ref[...] = acc

    return pl.pallas_call(
        body,
        grid=(n_blocks,),
        in_specs=[
            pl.BlockSpec((D, TBLK), lambda j: (0, j)),
            pl.BlockSpec((128, 128), lambda j: (0, 0)),
        ],
        out_specs=pl.BlockSpec((TSUB, 128), lambda j: (j, 0)),
        out_shape=jax.ShapeDtypeStruct((n_blocks * TSUB, 128), jnp.float32),
        compiler_params=pltpu.CompilerParams(fuse_transposed_lhs_in_matmul=True),
    )(table_t, ident)


def _sc_gather(x_grouped, table, L, B):
    n_idx_rows = x_grouped.shape[1]
    n_per_w = n_idx_rows * IDX_MINOR
    n_chunks = n_per_w // CHUNK
    kb_per_l = B // KB

    mesh = plsc.VectorSubcoreMesh(core_axis_name="c", subcore_axis_name="s")

    @functools.partial(
        pl.kernel,
        mesh=mesh,
        out_type=jax.ShapeDtypeStruct((L, kb_per_l, RB, 4, D), jnp.float32),
        compiler_params=pltpu.CompilerParams(use_tc_tiling_on_sc=False),
        scratch_types=[
            pltpu.VMEM((n_idx_rows, IDX_MINOR), jnp.int32),
            pltpu.VMEM((CHUNK, D), jnp.float32),
            pltpu.SemaphoreType.DMA,
        ],
    )
    def k(x_hbm, table_hbm, out_hbm, idx_v, rows_v, sem):
        wid = lax.axis_index("c") * NS + lax.axis_index("s")
        ch0 = wid * n_chunks
        pltpu.sync_copy(x_hbm.at[wid], idx_v)

        @pl.loop(0, n_chunks, unroll=1)
        def chunk_body(c):
            ch = ch0 + c
            span = ch // 2
            uc = ch % 2
            l = span // kb_per_l
            kb = span % kb_per_l
            descs = []
            for j in range(GATHERS_PER_CHUNK):
                descs.append(
                    pltpu.async_copy(
                        table_hbm.at[idx_v.at[c * GATHERS_PER_CHUNK + j]],
                        rows_v.at[pl.ds(j * IDX_MINOR, IDX_MINOR)],
                        sem,
                    )
                )
            for dsc in descs:
                dsc.wait()
            pltpu.sync_copy(rows_v.at[pl.ds(0, RB)], out_hbm.at[l, kb, :, 2 * uc])
            pltpu.sync_copy(
                rows_v.at[pl.ds(RB, RB)], out_hbm.at[l, kb, :, 2 * uc + 1]
            )

    return k(x_grouped, table)


def _tc_norm_t(z4, seg, gd, bb, L, B):
    nkb = B // KB

    def body(z_ref, seg_ref, g_ref, b_ref, o_ref):
        z = z_ref[...]
        s = seg_ref[...]
        gd32 = g_ref[...]
        bcol = b_ref[...][:, 0:1]
        s1 = jnp.dot(z, s, preferred_element_type=jnp.float32)
        s2 = jnp.dot(z * z, s, preferred_element_type=jnp.float32)
        mean = s1 * (1.0 / D)
        var = s2 * (1.0 / D) - mean * mean
        rstd = lax.rsqrt(var + EPS)
        u = (z - mean) * rstd
        for sseg in range(4):
            uq = u[:, sseg * D:(sseg + 1) * D]
            t = lax.dot_general(
                gd32, uq, (((1,), (1,)), ((), ())),
                precision=lax.Precision.HIGHEST,
                preferred_element_type=jnp.float32,
            )
            o_ref[0, :, sseg * RB:(sseg + 1) * RB] = jnp.maximum(t + bcol, 0.0)

    return pl.pallas_call(
        body,
        grid=(L, nkb),
        in_specs=[
            pl.BlockSpec((RB, 128), lambda l, kb: (l * nkb + kb, 0)),
            pl.BlockSpec((128, 128), lambda l, kb: (0, 0)),
            pl.BlockSpec((D, D), lambda l, kb: (0, 0)),
            pl.BlockSpec((D, 128), lambda l, kb: (0, 0)),
        ],
        out_specs=pl.BlockSpec((1, D, KB), lambda l, kb: (l, 0, kb)),
        out_shape=jax.ShapeDtypeStruct((L, D, B), jnp.float32),
    )(z4, seg, gd, bb)


def kernel(x, table, gamma, beta):
    B, L = x.shape
    V = table.shape[0]
    N = B * L
    n_blocks = (V + TBLK - 1) // TBLK

    ident = jnp.asarray(np.eye(128), dtype=jnp.float32)
    t4 = _tc_pack_table(table.T, ident, n_blocks)
    tpack = t4.reshape(n_blocks * TBLK, D)

    xq = (
        jnp.left_shift(jnp.right_shift(x, 13), 13)
        | jnp.left_shift(x & 2047, 2)
        | (jnp.right_shift(x, 11) & 3)
    )
    xp = xq.T.reshape(NW, N // (NW * IDX_MINOR), IDX_MINOR)
    g = _sc_gather(xp, tpack, L, B)

    z4 = g.reshape(N // 4, 4 * D)
    seg = jnp.asarray(
        (np.arange(128)[:, None] // D) == (np.arange(128)[None, :] // D),
        dtype=jnp.float32,
    )
    gd = jnp.diag(gamma)
    bb = jnp.broadcast_to(beta[:, None], (D, 128))
    out_t = _tc_norm_t(z4, seg, gd, bb, L, B)
    return jnp.transpose(out_t, (2, 0, 1))


# speedup vs baseline: 1.5990x; 1.5990x over previous
import functools

import jax
import jax.numpy as jnp
import numpy as np
from jax import lax
from jax.experimental import pallas as pl
from jax.experimental.pallas import tpu as pltpu
from jax.experimental.pallas import tpu_sc as plsc

D = 32
EPS = 1e-5

NC = 2
NS = 16
NW = NC * NS

IDX_MINOR = 128
GATHERS_PER_CHUNK = 8
CHUNK = IDX_MINOR * GATHERS_PER_CHUNK

TBLK = 8192
TSUB = TBLK // 4

KB = 2048
RB = KB // 4


def _tc_pack_table(table_t, ident, n_blocks):
    def body(z_ref, e_ref, o_ref):
        z = z_ref[...]
        acc = None
        for q in range(4):
            zq = z[:, q * TSUB:(q + 1) * TSUB]
            eq = e_ref[pl.ds(q * D, D), :]
            zh = zq.astype(jnp.bfloat16).astype(jnp.float32)
            zl = zq - zh
            t = lax.dot_general(
                zh, eq, (((0,), (0,)), ((), ())),
                preferred_element_type=jnp.float32,
            ) + lax.dot_general(
                zl, eq, (((0,), (0,)), ((), ())),
                preferred_element_type=jnp.float32,
            )
            acc = t if acc is None else acc + t
        o_ref[...] = acc

    return pl.pallas_call(
        body,
        grid=(n_blocks,),
        in_specs=[
            pl.BlockSpec((D, TBLK), lambda j: (0, j)),
            pl.BlockSpec((128, 128), lambda j: (0, 0)),
        ],
        out_specs=pl.BlockSpec((TSUB, 128), lambda j: (j, 0)),
        out_shape=jax.ShapeDtypeStruct((n_blocks * TSUB, 128), jnp.float32),
        compiler_params=pltpu.CompilerParams(fuse_transposed_lhs_in_matmul=True),
    )(table_t, ident)


def _sc_gather(x_grouped, table, L, B):
    n_idx_rows = x_grouped.shape[1]
    n_per_w = n_idx_rows * IDX_MINOR
    n_chunks = n_per_w // CHUNK
    kb_per_l = B // KB

    mesh = plsc.VectorSubcoreMesh(core_axis_name="c", subcore_axis_name="s")

    @functools.partial(
        pl.kernel,
        mesh=mesh,
        out_type=jax.ShapeDtypeStruct((L, kb_per_l, RB, 4, D), jnp.float32),
        compiler_params=pltpu.CompilerParams(use_tc_tiling_on_sc=False),
        scratch_types=[
            pltpu.VMEM((n_idx_rows, IDX_MINOR), jnp.int32),
            pltpu.VMEM((CHUNK, D), jnp.float32),
            pltpu.SemaphoreType.DMA,
        ],
    )
    def k(x_hbm, table_hbm, out_hbm, idx_v, rows_v, sem):
        wid = lax.axis_index("c") * NS + lax.axis_index("s")
        ch0 = wid * n_chunks
        pltpu.sync_copy(x_hbm.at[wid], idx_v)

        @pl.loop(0, n_chunks, unroll=1)
        def chunk_body(c):
            ch = ch0 + c
            span = ch // 2
            uc = ch % 2
            l = span // kb_per_l
            kb = span % kb_per_l
            descs = []
            for j in range(GATHERS_PER_CHUNK):
                descs.append(
                    pltpu.async_copy(
                        table_hbm.at[idx_v.at[c * GATHERS_PER_CHUNK + j]],
                        rows_v.at[pl.ds(j * IDX_MINOR, IDX_MINOR)],
                        sem,
                    )
                )
            for dsc in descs:
                dsc.wait()
            pltpu.sync_copy(rows_v.at[pl.ds(0, RB)], out_hbm.at[l, kb, :, 2 * uc])
            pltpu.sync_copy(
                rows_v.at[pl.ds(RB, RB)], out_hbm.at[l, kb, :, 2 * uc + 1]
            )

    return k(x_grouped, table)


def _tc_norm_t(z4, seg, gd, bb, L, B):
    nkb = B // KB

    def body(z_ref, seg_ref, g_ref, b_ref, o_ref):
        z = z_ref[...]
        s = seg_ref[...]
        gd32 = g_ref[...]
        bcol = b_ref[...][:, 0:1]
        s1 = jnp.dot(z, s, preferred_element_type=jnp.float32)
        s2 = jnp.dot(z * z, s, preferred_element_type=jnp.float32)
        mean = s1 * (1.0 / D)
        var = s2 * (1.0 / D) - mean * mean
        rstd = lax.rsqrt(var + EPS)
        u = (z - mean) * rstd
        for sseg in range(4):
            uq = u[:, sseg * D:(sseg + 1) * D]
            uh = uq.astype(jnp.bfloat16).astype(jnp.float32)
            ul = uq - uh
            t = lax.dot_general(
                gd32, uh, (((1,), (1,)), ((), ())),
                preferred_element_type=jnp.float32,
            ) + lax.dot_general(
                gd32, ul, (((1,), (1,)), ((), ())),
                preferred_element_type=jnp.float32,
            )
            o_ref[0, :, sseg * RB:(sseg + 1) * RB] = jnp.maximum(t + bcol, 0.0)

    return pl.pallas_call(
        body,
        grid=(L, nkb),
        in_specs=[
            pl.BlockSpec((RB, 128), lambda l, kb: (l * nkb + kb, 0)),
            pl.BlockSpec((128, 128), lambda l, kb: (0, 0)),
            pl.BlockSpec((D, D), lambda l, kb: (0, 0)),
            pl.BlockSpec((D, 128), lambda l, kb: (0, 0)),
        ],
        out_specs=pl.BlockSpec((1, D, KB), lambda l, kb: (l, 0, kb)),
        out_shape=jax.ShapeDtypeStruct((L, D, B), jnp.float32),
    )(z4, seg, gd, bb)


def kernel(x, table, gamma, beta):
    B, L = x.shape
    V = table.shape[0]
    N = B * L
    n_blocks = (V + TBLK - 1) // TBLK

    ident = jnp.asarray(np.eye(128), dtype=jnp.float32)
    t4 = _tc_pack_table(table.T, ident, n_blocks)
    tpack = t4.reshape(n_blocks * TBLK, D)

    xq = (
        jnp.left_shift(jnp.right_shift(x, 13), 13)
        | jnp.left_shift(x & 2047, 2)
        | (jnp.right_shift(x, 11) & 3)
    )
    xp = xq.T.reshape(NW, N // (NW * IDX_MINOR), IDX_MINOR)
    g = _sc_gather(xp, tpack, L, B)

    z4 = g.reshape(N // 4, 4 * D)
    seg = jnp.asarray(
        (np.arange(128)[:, None] // D) == (np.arange(128)[None, :] // D),
        dtype=jnp.float32,
    )
    gd = jnp.diag(gamma)
    bb = jnp.broadcast_to(beta[:, None], (D, 128))
    out_t = _tc_norm_t(z4, seg, gd, bb, L, B)
    return jnp.transpose(out_t, (2, 0, 1))


# speedup vs baseline: 1.8322x; 1.1459x over previous
import functools

import jax
import jax.numpy as jnp
import numpy as np
from jax import lax
from jax.experimental import pallas as pl
from jax.experimental.pallas import tpu as pltpu
from jax.experimental.pallas import tpu_sc as plsc

D = 32
EPS = 1e-5

NC = 2
NS = 16
NW = NC * NS

IDX_MINOR = 128
GATHERS_PER_CHUNK = 8
CHUNK = IDX_MINOR * GATHERS_PER_CHUNK

TBLK = 8192
TSUB = TBLK // 4

KB = 2048
RB = KB // 4


def _tc_pack_table(table_t, ident, n_blocks):
    def body(z_ref, e_ref, o_ref):
        z = z_ref[...]
        acc = None
        for q in range(4):
            zq = z[:, q * TSUB:(q + 1) * TSUB]
            eq = e_ref[pl.ds(q * D, D), :]
            t = lax.dot_general(
                zq, eq, (((0,), (0,)), ((), ())),
                preferred_element_type=jnp.float32,
            )
            acc = t if acc is None else acc + t
        o_ref[...] = acc

    return pl.pallas_call(
        body,
        grid=(n_blocks,),
        in_specs=[
            pl.BlockSpec((D, TBLK), lambda j: (0, j)),
            pl.BlockSpec((128, 128), lambda j: (0, 0)),
        ],
        out_specs=pl.BlockSpec((TSUB, 128), lambda j: (j, 0)),
        out_shape=jax.ShapeDtypeStruct((n_blocks * TSUB, 128), jnp.float32),
        compiler_params=pltpu.CompilerParams(fuse_transposed_lhs_in_matmul=True),
    )(table_t, ident)


def _sc_gather(x_grouped, table, L, B):
    n_idx_rows = x_grouped.shape[1]
    n_per_w = n_idx_rows * IDX_MINOR
    n_chunks = n_per_w // CHUNK
    kb_per_l = B // KB

    mesh = plsc.VectorSubcoreMesh(core_axis_name="c", subcore_axis_name="s")

    @functools.partial(
        pl.kernel,
        mesh=mesh,
        out_type=jax.ShapeDtypeStruct((L, kb_per_l, RB, 4, D), jnp.float32),
        compiler_params=pltpu.CompilerParams(use_tc_tiling_on_sc=False),
        scratch_types=[
            pltpu.VMEM((n_idx_rows, IDX_MINOR), jnp.int32),
            pltpu.VMEM((CHUNK, D), jnp.float32),
            pltpu.SemaphoreType.DMA,
        ],
    )
    def k(x_hbm, table_hbm, out_hbm, idx_v, rows_v, sem):
        wid = lax.axis_index("c") * NS + lax.axis_index("s")
        ch0 = wid * n_chunks
        pltpu.sync_copy(x_hbm.at[wid], idx_v)

        @pl.loop(0, n_chunks, unroll=1)
        def chunk_body(c):
            ch = ch0 + c
            span = ch // 2
            uc = ch % 2
            l = span // kb_per_l
            kb = span % kb_per_l
            descs = []
            for j in range(GATHERS_PER_CHUNK):
                descs.append(
                    pltpu.async_copy(
                        table_hbm.at[idx_v.at[c * GATHERS_PER_CHUNK + j]],
                        rows_v.at[pl.ds(j * IDX_MINOR, IDX_MINOR)],
                        sem,
                    )
                )
            for dsc in descs:
                dsc.wait()
            pltpu.sync_copy(rows_v.at[pl.ds(0, RB)], out_hbm.at[l, kb, :, 2 * uc])
            pltpu.sync_copy(
                rows_v.at[pl.ds(RB, RB)], out_hbm.at[l, kb, :, 2 * uc + 1]
            )

    return k(x_grouped, table)


def _tc_norm_t(z4, seg, gd, bb, L, B):
    nkb = B // KB

    def body(z_ref, seg_ref, g_ref, b_ref, o_ref):
        z = z_ref[...]
        s = seg_ref[...]
        gd32 = g_ref[...]
        bcol = b_ref[...][:, 0:1]
        s1 = jnp.dot(z, s, preferred_element_type=jnp.float32)
        s2 = jnp.dot(z * z, s, preferred_element_type=jnp.float32)
        mean = s1 * (1.0 / D)
        var = s2 * (1.0 / D) - mean * mean
        rstd = lax.rsqrt(var + EPS)
        u = (z - mean) * rstd
        for sseg in range(4):
            uq = u[:, sseg * D:(sseg + 1) * D]
            t = lax.dot_general(
                gd32, uq, (((1,), (1,)), ((), ())),
                preferred_element_type=jnp.float32,
            )
            o_ref[0, :, sseg * RB:(sseg + 1) * RB] = jnp.maximum(t + bcol, 0.0)

    return pl.pallas_call(
        body,
        grid=(L, nkb),
        in_specs=[
            pl.BlockSpec((RB, 128), lambda l, kb: (l * nkb + kb, 0)),
            pl.BlockSpec((128, 128), lambda l, kb: (0, 0)),
            pl.BlockSpec((D, D), lambda l, kb: (0, 0)),
            pl.BlockSpec((D, 128), lambda l, kb: (0, 0)),
        ],
        out_specs=pl.BlockSpec((1, D, KB), lambda l, kb: (l, 0, kb)),
        out_shape=jax.ShapeDtypeStruct((L, D, B), jnp.float32),
    )(z4, seg, gd, bb)


def kernel(x, table, gamma, beta):
    B, L = x.shape
    V = table.shape[0]
    N = B * L
    n_blocks = (V + TBLK - 1) // TBLK

    ident = jnp.asarray(np.eye(128), dtype=jnp.float32)
    t4 = _tc_pack_table(table.T, ident, n_blocks)
    tpack = t4.reshape(n_blocks * TBLK, D)

    xq = (
        jnp.left_shift(jnp.right_shift(x, 13), 13)
        | jnp.left_shift(x & 2047, 2)
        | (jnp.right_shift(x, 11) & 3)
    )
    xp = xq.T.reshape(NW, N // (NW * IDX_MINOR), IDX_MINOR)
    g = _sc_gather(xp, tpack, L, B)

    z4 = g.reshape(N // 4, 4 * D)
    seg = jnp.asarray(
        (np.arange(128)[:, None] // D) == (np.arange(128)[None, :] // D),
        dtype=jnp.float32,
    )
    gd = jnp.diag(gamma)
    bb = jnp.broadcast_to(beta[:, None], (D, 128))
    out_t = _tc_norm_t(z4, seg, gd, bb, L, B)
    return jnp.transpose(out_t, (2, 0, 1))
